# baseline (speedup 1.0000x reference)
import functools

import jax
import jax.numpy as jnp
from jax import lax
from jax.experimental import pallas as pl
from jax.experimental.pallas import tpu as pltpu
from jax.experimental.pallas import tpu_sc as plsc

N = 10000
E = 320000
NPAD = 10112
CHUNK = 512
CPT = 20
EPT = CPT * CHUNK
TILES = 32
EPAD = TILES * EPT
DEGW = 16
NBUF = 4
ROWS_PER_TILE = NPAD // 16

_MESH = dict(core_axis_name="c", subcore_axis_name="s")
_SC_PARAMS = pltpu.CompilerParams(use_tc_tiling_on_sc=False)


@functools.partial(
    pl.kernel,
    out_type=jax.ShapeDtypeStruct((2, NPAD, DEGW), jnp.float32),
    mesh=plsc.VectorSubcoreMesh(**_MESH),
    scratch_types=[
        pltpu.VMEM((CPT, CHUNK), jnp.int32),
        pltpu.VMEM((CHUNK, DEGW), jnp.float32),
        pltpu.VMEM_SHARED((NPAD, DEGW), jnp.float32),
        pltpu.SemaphoreType.DMA,
    ],
    compiler_params=_SC_PARAMS,
)
def _deg_kernel(dst_hbm, ones_hbm, zeros_hbm, out_hbm, dst_v, ones_v, acc_sh, sem):
    cid = lax.axis_index("c")
    sid = lax.axis_index("s")
    wid = cid * 16 + sid
    pltpu.sync_copy(dst_hbm.at[wid], dst_v)
    pltpu.sync_copy(ones_hbm, ones_v)
    row0 = sid * ROWS_PER_TILE
    pltpu.sync_copy(zeros_hbm.at[pl.ds(row0, ROWS_PER_TILE)],
                    acc_sh.at[pl.ds(row0, ROWS_PER_TILE)])
    plsc.subcore_barrier()

    def fire(j, carry):
        pltpu.async_copy(ones_v, acc_sh.at[dst_v.at[j]], sem, add=True)
        return carry

    lax.fori_loop(0, CPT, fire, 0)

    def drain(j, carry):
        pltpu.make_async_copy(ones_v, acc_sh.at[dst_v.at[0]], sem).wait()
        return carry

    lax.fori_loop(0, CPT, drain, 0)
    plsc.subcore_barrier()
    pltpu.sync_copy(acc_sh.at[pl.ds(row0, ROWS_PER_TILE)],
                    out_hbm.at[cid, pl.ds(row0, ROWS_PER_TILE)])


def _make_scatter(D):
    @functools.partial(
        pl.kernel,
        out_type=jax.ShapeDtypeStruct((2, NPAD, D), jnp.float32),
        mesh=plsc.VectorSubcoreMesh(**_MESH),
        scratch_types=(
            [pltpu.VMEM((CPT, CHUNK), jnp.int32),
             pltpu.VMEM((CPT, CHUNK), jnp.int32)]
            + [pltpu.VMEM((CHUNK, D), jnp.float32) for _ in range(NBUF)]
            + [pltpu.VMEM_SHARED((NPAD, D), jnp.float32)]
            + [pltpu.SemaphoreType.DMA for _ in range(2 * NBUF)]
        ),
        compiler_params=_SC_PARAMS,
    )
    def _scat(src_hbm, dst_hbm, y_hbm, zeros_hbm, out_hbm,
              src_v, dst_v, *rest):
        bufs = rest[:NBUF]
        acc_sh = rest[NBUF]
        gsem = rest[NBUF + 1:NBUF + 1 + NBUF]
        ssem = rest[NBUF + 1 + NBUF:]
        cid = lax.axis_index("c")
        sid = lax.axis_index("s")
        wid = cid * 16 + sid
        pltpu.sync_copy(src_hbm.at[wid], src_v)
        pltpu.sync_copy(dst_hbm.at[wid], dst_v)
        row0 = sid * ROWS_PER_TILE
        pltpu.sync_copy(zeros_hbm.at[pl.ds(row0, ROWS_PER_TILE)],
                        acc_sh.at[pl.ds(row0, ROWS_PER_TILE)])
        plsc.subcore_barrier()

        for b in range(NBUF):
            pltpu.async_copy(y_hbm.at[src_v.at[b]], bufs[b], gsem[b])

        def body(o, carry):
            j0 = o * NBUF
            for b in range(NBUF):
                j = j0 + b
                pltpu.make_async_copy(
                    y_hbm.at[src_v.at[j]], bufs[b], gsem[b]).wait()
                pltpu.async_copy(
                    bufs[b], acc_sh.at[dst_v.at[j]], ssem[b], add=True)
            for b in range(NBUF):
                j = j0 + b

                @pl.when(j + NBUF < CPT)
                def _():
                    pltpu.make_async_copy(
                        bufs[b], acc_sh.at[dst_v.at[j]], ssem[b]).wait()
                    pltpu.async_copy(
                        y_hbm.at[src_v.at[j + NBUF]], bufs[b], gsem[b])
            return carry

        lax.fori_loop(0, CPT // NBUF, body, 0)
        for b in range(NBUF):
            pltpu.make_async_copy(
                bufs[b], acc_sh.at[dst_v.at[CPT - NBUF + b]], ssem[b]).wait()
        plsc.subcore_barrier()
        pltpu.sync_copy(acc_sh.at[pl.ds(row0, ROWS_PER_TILE)],
                        out_hbm.at[cid, pl.ds(row0, ROWS_PER_TILE)])

    return _scat


_scat32 = _make_scatter(32)
_scat16 = _make_scatter(16)


_RB = 2000


def _dis_block(degp_ref):
    deg = degp_ref[0][:, 0:1] + degp_ref[1][:, 0:1] + 1.0
    return lax.rsqrt(deg)


def _tcA_body(x_ref, w_ref, degp_ref, y_ref):
    xw = jnp.dot(x_ref[...], w_ref[...], preferred_element_type=jnp.float32)
    y_ref[...] = xw * _dis_block(degp_ref)


_tcA = pl.pallas_call(
    _tcA_body,
    grid=(N // _RB,),
    in_specs=[
        pl.BlockSpec((_RB, 128), lambda i: (i, 0)),
        pl.BlockSpec((128, 32), lambda i: (0, 0)),
        pl.BlockSpec((2, _RB, DEGW), lambda i: (0, i, 0)),
    ],
    out_specs=pl.BlockSpec((_RB, 32), lambda i: (i, 0)),
    out_shape=jax.ShapeDtypeStruct((N, 32), jnp.float32),
)


def _tcB_body(s_ref, y1_ref, degp_ref, b1_ref, w2_ref, y2_ref):
    dis = _dis_block(degp_ref)
    h = dis * (s_ref[0] + s_ref[1] + y1_ref[...]) + b1_ref[...]
    h = jnp.maximum(h, 0.0)
    y2_ref[...] = jnp.dot(h, w2_ref[...],
                          preferred_element_type=jnp.float32) * dis


_tcB = pl.pallas_call(
    _tcB_body,
    grid=(N // _RB,),
    in_specs=[
        pl.BlockSpec((2, _RB, 32), lambda i: (0, i, 0)),
        pl.BlockSpec((_RB, 32), lambda i: (i, 0)),
        pl.BlockSpec((2, _RB, DEGW), lambda i: (0, i, 0)),
        pl.BlockSpec((1, 32), lambda i: (0, 0)),
        pl.BlockSpec((32, 16), lambda i: (0, 0)),
    ],
    out_specs=pl.BlockSpec((_RB, 16), lambda i: (i, 0)),
    out_shape=jax.ShapeDtypeStruct((N, 16), jnp.float32),
)


def _tcC_body(s_ref, y2_ref, degp_ref, b2_ref, w3_ref, b3_ref, out_ref):
    dis = _dis_block(degp_ref)
    h = dis * (s_ref[0] + s_ref[1] + y2_ref[...]) + b2_ref[...]
    h = jnp.maximum(h, 0.0)
    out_ref[...] = jnp.dot(h, w3_ref[...],
                           preferred_element_type=jnp.float32) + b3_ref[...]


_tcC = pl.pallas_call(
    _tcC_body,
    grid=(N // _RB,),
    in_specs=[
        pl.BlockSpec((2, _RB, 16), lambda i: (0, i, 0)),
        pl.BlockSpec((_RB, 16), lambda i: (i, 0)),
        pl.BlockSpec((2, _RB, DEGW), lambda i: (0, i, 0)),
        pl.BlockSpec((1, 16), lambda i: (0, 0)),
        pl.BlockSpec((16, 2), lambda i: (0, 0)),
        pl.BlockSpec((1, 2), lambda i: (0, 0)),
    ],
    out_specs=pl.BlockSpec((_RB, 2), lambda i: (i, 0)),
    out_shape=jax.ShapeDtypeStruct((N, 2), jnp.float32),
)


def kernel(x, edge_index, W1, b1, W2, b2, W3, b3):
    src = edge_index[0].astype(jnp.int32)
    dst = edge_index[1].astype(jnp.int32)
    src_p = jnp.concatenate(
        [src, jnp.zeros((EPAD - E,), jnp.int32)]).reshape(TILES, CPT, CHUNK)
    dst_p = jnp.concatenate(
        [dst, jnp.full((EPAD - E,), N, jnp.int32)]).reshape(TILES, CPT, CHUNK)

    ones_deg = jnp.ones((CHUNK, DEGW), jnp.float32)
    zeros_deg = jnp.zeros((NPAD, DEGW), jnp.float32)
    zeros32 = jnp.zeros((NPAD, 32), jnp.float32)
    zeros16 = jnp.zeros((NPAD, 16), jnp.float32)

    degp = _deg_kernel(dst_p, ones_deg, zeros_deg)
    y1 = _tcA(x, W1, degp)
    s1 = _scat32(src_p, dst_p, y1, zeros32)
    y2 = _tcB(s1, y1, degp, b1.reshape(1, 32), W2)
    s2 = _scat16(src_p, dst_p, y2, zeros16)
    out = _tcC(s2, y2, degp, b2.reshape(1, 16), W3, b3.reshape(1, 2))
    return out


# speedup vs baseline: 1.9485x; 1.9485x over previous
import functools

import jax
import jax.numpy as jnp
from jax import lax
from jax.experimental import pallas as pl
from jax.experimental.pallas import tpu as pltpu
from jax.experimental.pallas import tpu_sc as plsc

N = 10000
E = 320000
NPAD = 10112
CHUNK = 1000
CPT = 10
EPT = CPT * CHUNK
TILES = 32
DEGW = 16
NBUF = 2
ROWS_PER_TILE = NPAD // 16

_MESH = dict(core_axis_name="c", subcore_axis_name="s")
_SC_PARAMS = pltpu.CompilerParams(use_tc_tiling_on_sc=False)


@functools.partial(
    pl.kernel,
    out_type=jax.ShapeDtypeStruct((2, NPAD, DEGW), jnp.float32),
    mesh=plsc.VectorSubcoreMesh(**_MESH),
    scratch_types=[
        pltpu.VMEM((EPT,), jnp.int32),
        pltpu.VMEM((CHUNK, DEGW), jnp.float32),
        pltpu.VMEM_SHARED((NPAD, DEGW), jnp.float32),
        pltpu.SemaphoreType.DMA,
    ],
    compiler_params=_SC_PARAMS,
)
def _deg_kernel(ei_hbm, ones_hbm, zeros_hbm, out_hbm, dst_v, ones_v, acc_sh, sem):
    cid = lax.axis_index("c")
    sid = lax.axis_index("s")
    wid = cid * 16 + sid
    pltpu.sync_copy(ei_hbm.at[1, pl.ds(wid * EPT, EPT)], dst_v)
    pltpu.sync_copy(ones_hbm, ones_v)
    row0 = sid * ROWS_PER_TILE
    pltpu.sync_copy(zeros_hbm.at[pl.ds(row0, ROWS_PER_TILE)],
                    acc_sh.at[pl.ds(row0, ROWS_PER_TILE)])
    plsc.subcore_barrier()

    def fire(j, carry):
        pltpu.async_copy(ones_v, acc_sh.at[dst_v.at[pl.ds(j * CHUNK, CHUNK)]],
                         sem, add=True)
        return carry

    lax.fori_loop(0, CPT, fire, 0)

    def drain(j, carry):
        pltpu.make_async_copy(ones_v, acc_sh.at[dst_v.at[pl.ds(0, CHUNK)]],
                              sem).wait()
        return carry

    lax.fori_loop(0, CPT, drain, 0)
    plsc.subcore_barrier()
    pltpu.sync_copy(acc_sh.at[pl.ds(row0, ROWS_PER_TILE)],
                    out_hbm.at[cid, pl.ds(row0, ROWS_PER_TILE)])


def _make_scatter(D):
    @functools.partial(
        pl.kernel,
        out_type=jax.ShapeDtypeStruct((2, NPAD, D), jnp.float32),
        mesh=plsc.VectorSubcoreMesh(**_MESH),
        scratch_types=(
            [pltpu.VMEM((EPT,), jnp.int32),
             pltpu.VMEM((EPT,), jnp.int32)]
            + [pltpu.VMEM((CHUNK, D), jnp.float32) for _ in range(NBUF)]
            + [pltpu.VMEM_SHARED((NPAD, D), jnp.float32)]
            + [pltpu.SemaphoreType.DMA for _ in range(2 * NBUF)]
        ),
        compiler_params=_SC_PARAMS,
    )
    def _scat(ei_hbm, y_hbm, zeros_hbm, out_hbm, src_v, dst_v, *rest):
        bufs = rest[:NBUF]
        acc_sh = rest[NBUF]
        gsem = rest[NBUF + 1:NBUF + 1 + NBUF]
        ssem = rest[NBUF + 1 + NBUF:]
        cid = lax.axis_index("c")
        sid = lax.axis_index("s")
        wid = cid * 16 + sid
        pltpu.sync_copy(ei_hbm.at[0, pl.ds(wid * EPT, EPT)], src_v)
        pltpu.sync_copy(ei_hbm.at[1, pl.ds(wid * EPT, EPT)], dst_v)
        row0 = sid * ROWS_PER_TILE
        pltpu.sync_copy(zeros_hbm.at[pl.ds(row0, ROWS_PER_TILE)],
                        acc_sh.at[pl.ds(row0, ROWS_PER_TILE)])
        plsc.subcore_barrier()

        def sidx(j):
            return src_v.at[pl.ds(j * CHUNK, CHUNK)]

        def didx(j):
            return dst_v.at[pl.ds(j * CHUNK, CHUNK)]

        for b in range(NBUF):
            pltpu.async_copy(y_hbm.at[sidx(b)], bufs[b], gsem[b])

        def body(o, carry):
            j0 = o * NBUF
            for b in range(NBUF):
                j = j0 + b
                pltpu.make_async_copy(y_hbm.at[sidx(j)], bufs[b],
                                      gsem[b]).wait()
                pltpu.async_copy(bufs[b], acc_sh.at[didx(j)], ssem[b],
                                 add=True)
            for b in range(NBUF):
                j = j0 + b

                @pl.when(j + NBUF < CPT)
                def _():
                    pltpu.make_async_copy(bufs[b], acc_sh.at[didx(j)],
                                          ssem[b]).wait()
                    pltpu.async_copy(y_hbm.at[sidx(j + NBUF)], bufs[b],
                                     gsem[b])
            return carry

        lax.fori_loop(0, CPT // NBUF, body, 0)
        for b in range(NBUF):
            pltpu.make_async_copy(bufs[b], acc_sh.at[didx(CPT - NBUF + b)],
                                  ssem[b]).wait()
        plsc.subcore_barrier()
        pltpu.sync_copy(acc_sh.at[pl.ds(row0, ROWS_PER_TILE)],
                        out_hbm.at[cid, pl.ds(row0, ROWS_PER_TILE)])

    return _scat


_scat32 = _make_scatter(32)
_scat16 = _make_scatter(16)


def _dis_full(degp_ref):
    deg = degp_ref[0, :, 0:1] + degp_ref[1, :, 0:1] + 1.0
    return lax.rsqrt(deg)[: N]


def _tcxw_body(x_ref, w_ref, xw_ref):
    xw_ref[...] = jnp.dot(x_ref[...], w_ref[...],
                          preferred_element_type=jnp.float32)


_tcxw = pl.pallas_call(
    _tcxw_body,
    out_shape=jax.ShapeDtypeStruct((N, 32), jnp.float32),
)


def _tcscale_body(xw_ref, degp_ref, y_ref):
    y_ref[...] = xw_ref[...] * _dis_full(degp_ref)


_tcscale = pl.pallas_call(
    _tcscale_body,
    out_shape=jax.ShapeDtypeStruct((N, 32), jnp.float32),
)


def _tcB_body(s_ref, y1_ref, degp_ref, b1_ref, w2_ref, y2_ref):
    dis = _dis_full(degp_ref)
    h = dis * (s_ref[0, :N] + s_ref[1, :N] + y1_ref[...]) + b1_ref[...]
    h = jnp.maximum(h, 0.0)
    y2_ref[...] = jnp.dot(h, w2_ref[...],
                          preferred_element_type=jnp.float32) * dis


_tcB = pl.pallas_call(
    _tcB_body,
    out_shape=jax.ShapeDtypeStruct((N, 16), jnp.float32),
)


def _tcC_body(s_ref, y2_ref, degp_ref, b2_ref, w3_ref, b3_ref, out_ref):
    dis = _dis_full(degp_ref)
    h = dis * (s_ref[0, :N] + s_ref[1, :N] + y2_ref[...]) + b2_ref[...]
    h = jnp.maximum(h, 0.0)
    out_ref[...] = jnp.dot(h, w3_ref[...],
                           preferred_element_type=jnp.float32) + b3_ref[...]


_tcC = pl.pallas_call(
    _tcC_body,
    out_shape=jax.ShapeDtypeStruct((N, 2), jnp.float32),
)


def kernel(x, edge_index, W1, b1, W2, b2, W3, b3):
    ei = edge_index.astype(jnp.int32)

    ones_deg = jnp.ones((CHUNK, DEGW), jnp.float32)
    zeros_deg = jnp.zeros((NPAD, DEGW), jnp.float32)
    zeros32 = jnp.zeros((NPAD, 32), jnp.float32)
    zeros16 = jnp.zeros((NPAD, 16), jnp.float32)

    degp = _deg_kernel(ei, ones_deg, zeros_deg)
    xw = _tcxw(x, W1)
    y1 = _tcscale(xw, degp)
    s1 = _scat32(ei, y1, zeros32)
    y2 = _tcB(s1, y1, degp, b1.reshape(1, 32), W2)
    s2 = _scat16(ei, y2, zeros16)
    out = _tcC(s2, y2, degp, b2.reshape(1, 16), W3, b3.reshape(1, 2))
    return out


# speedup vs baseline: 2.0730x; 1.0639x over previous
import functools

import jax
import jax.numpy as jnp
from jax import lax
from jax.experimental import pallas as pl
from jax.experimental.pallas import tpu as pltpu
from jax.experimental.pallas import tpu_sc as plsc

N = 10000
E = 320000
NPAD = 10240
CHUNK = 1000
CPT = 10
EPT = CPT * CHUNK
TILES = 32
DEGW = 16
NBUF = 2
RPT = NPAD // 16

_MESH = dict(core_axis_name="c", subcore_axis_name="s")
_SC_PARAMS = pltpu.CompilerParams(use_tc_tiling_on_sc=False)


@functools.partial(
    pl.kernel,
    out_type=jax.ShapeDtypeStruct((NPAD, 128), jnp.float32),
    mesh=plsc.VectorSubcoreMesh(**_MESH),
    scratch_types=[
        pltpu.VMEM((EPT,), jnp.int32),
        pltpu.VMEM((CHUNK, DEGW), jnp.float32),
        pltpu.VMEM_SHARED((NPAD, DEGW), jnp.float32),
        pltpu.SemaphoreType.DMA,
    ],
    compiler_params=_SC_PARAMS,
)
def _deg_kernel(ei_hbm, ones_hbm, zeros_hbm, out_hbm, dst_v, ones_v, acc_sh, sem):
    cid = lax.axis_index("c")
    sid = lax.axis_index("s")
    wid = cid * 16 + sid
    pltpu.sync_copy(ei_hbm.at[1, pl.ds(wid * EPT, EPT)], dst_v)
    pltpu.sync_copy(ones_hbm, ones_v)
    row0 = sid * RPT
    pltpu.sync_copy(zeros_hbm.at[pl.ds(row0, RPT)],
                    acc_sh.at[pl.ds(row0, RPT)])
    plsc.subcore_barrier()

    def fire(j, carry):
        pltpu.async_copy(ones_v, acc_sh.at[dst_v.at[pl.ds(j * CHUNK, CHUNK)]],
                         sem, add=True)
        return carry

    lax.fori_loop(0, CPT, fire, 0)

    def drain(j, carry):
        pltpu.make_async_copy(ones_v, acc_sh.at[dst_v.at[pl.ds(0, CHUNK)]],
                              sem).wait()
        return carry

    lax.fori_loop(0, CPT, drain, 0)
    plsc.subcore_barrier()
    pltpu.sync_copy(acc_sh.at[pl.ds(row0, RPT)],
                    out_hbm.at[pl.ds(row0, RPT), pl.ds(cid * DEGW, DEGW)])


def _make_scatter(D):
    @functools.partial(
        pl.kernel,
        out_type=jax.ShapeDtypeStruct((NPAD, 128), jnp.float32),
        mesh=plsc.VectorSubcoreMesh(**_MESH),
        scratch_types=(
            [pltpu.VMEM((EPT,), jnp.int32),
             pltpu.VMEM((EPT,), jnp.int32)]
            + [pltpu.VMEM((CHUNK, D), jnp.float32) for _ in range(NBUF)]
            + [pltpu.VMEM_SHARED((NPAD, D), jnp.float32)]
            + [pltpu.SemaphoreType.DMA for _ in range(2 * NBUF)]
        ),
        compiler_params=_SC_PARAMS,
    )
    def _scat(ei_hbm, y_hbm, zeros_hbm, out_hbm, src_v, dst_v, *rest):
        bufs = rest[:NBUF]
        acc_sh = rest[NBUF]
        gsem = rest[NBUF + 1:NBUF + 1 + NBUF]
        ssem = rest[NBUF + 1 + NBUF:]
        cid = lax.axis_index("c")
        sid = lax.axis_index("s")
        wid = cid * 16 + sid
        pltpu.sync_copy(ei_hbm.at[0, pl.ds(wid * EPT, EPT)], src_v)
        pltpu.sync_copy(ei_hbm.at[1, pl.ds(wid * EPT, EPT)], dst_v)
        row0 = sid * RPT
        pltpu.sync_copy(zeros_hbm.at[pl.ds(row0, RPT)],
                        acc_sh.at[pl.ds(row0, RPT)])
        plsc.subcore_barrier()

        def sidx(j):
            return src_v.at[pl.ds(j * CHUNK, CHUNK)]

        def didx(j):
            return dst_v.at[pl.ds(j * CHUNK, CHUNK)]

        def gsrc(j):
            return y_hbm.at[sidx(j)]

        for b in range(NBUF):
            pltpu.async_copy(gsrc(b), bufs[b], gsem[b])

        def body(o, carry):
            j0 = o * NBUF
            for b in range(NBUF):
                j = j0 + b
                pltpu.make_async_copy(gsrc(j), bufs[b], gsem[b]).wait()
                pltpu.async_copy(bufs[b], acc_sh.at[didx(j)], ssem[b],
                                 add=True)
            for b in range(NBUF):
                j = j0 + b

                @pl.when(j + NBUF < CPT)
                def _():
                    pltpu.make_async_copy(bufs[b], acc_sh.at[didx(j)],
                                          ssem[b]).wait()
                    pltpu.async_copy(gsrc(j + NBUF), bufs[b], gsem[b])
            return carry

        lax.fori_loop(0, CPT // NBUF, body, 0)
        for b in range(NBUF):
            pltpu.make_async_copy(bufs[b], acc_sh.at[didx(CPT - NBUF + b)],
                                  ssem[b]).wait()
        plsc.subcore_barrier()
        pltpu.sync_copy(acc_sh.at[pl.ds(row0, RPT)],
                        out_hbm.at[pl.ds(row0, RPT), pl.ds(cid * D, D)])

    return _scat


_scat32 = _make_scatter(32)
_scat16 = _make_scatter(16)


_RB = 1000
_GRID = (N // _RB,)
_row_spec128 = pl.BlockSpec((_RB, 128), lambda i: (i, 0))


def _dis_block(deg_ref, off):
    deg = deg_ref[:, off:off + 1] + deg_ref[:, off + DEGW:off + DEGW + 1] + 1.0
    return lax.rsqrt(deg)


def _tcxw_body(x_ref, w_ref, xw_ref):
    xw_ref[...] = jnp.dot(x_ref[...], w_ref[...],
                          preferred_element_type=jnp.float32)


_tcxw = pl.pallas_call(
    _tcxw_body,
    grid=_GRID,
    in_specs=[
        _row_spec128,
        pl.BlockSpec((128, 32), lambda i: (0, 0)),
    ],
    out_specs=pl.BlockSpec((_RB, 32), lambda i: (i, 0)),
    out_shape=jax.ShapeDtypeStruct((N, 32), jnp.float32),
)


def _tcscale_body(xw_ref, deg_ref, y_ref):
    y_ref[...] = xw_ref[...] * _dis_block(deg_ref, 0)


_tcscale = pl.pallas_call(
    _tcscale_body,
    grid=_GRID,
    in_specs=[
        pl.BlockSpec((_RB, 32), lambda i: (i, 0)),
        _row_spec128,
    ],
    out_specs=pl.BlockSpec((_RB, 32), lambda i: (i, 0)),
    out_shape=jax.ShapeDtypeStruct((N, 32), jnp.float32),
)


def _tcB_body(s_ref, xw_ref, deg_ref, b1_ref, w2_ref, y2_ref):
    dis = _dis_block(deg_ref, 0)
    h = dis * (s_ref[:, 0:32] + s_ref[:, 32:64]) \
        + (dis * dis) * xw_ref[...] + b1_ref[...]
    h = jnp.maximum(h, 0.0)
    y2_ref[...] = jnp.dot(h, w2_ref[...],
                          preferred_element_type=jnp.float32) * dis


_tcB = pl.pallas_call(
    _tcB_body,
    grid=_GRID,
    in_specs=[
        _row_spec128,
        pl.BlockSpec((_RB, 32), lambda i: (i, 0)),
        _row_spec128,
        pl.BlockSpec((1, 32), lambda i: (0, 0)),
        pl.BlockSpec((32, 16), lambda i: (0, 0)),
    ],
    out_specs=pl.BlockSpec((_RB, 16), lambda i: (i, 0)),
    out_shape=jax.ShapeDtypeStruct((N, 16), jnp.float32),
)


def _tcC_body(s_ref, y2_ref, deg_ref, b2_ref, w3_ref, b3_ref, out_ref):
    dis = _dis_block(deg_ref, 0)
    h = dis * (s_ref[:, 0:16] + s_ref[:, 16:32] + y2_ref[...]) \
        + b2_ref[...]
    h = jnp.maximum(h, 0.0)
    out_ref[...] = jnp.dot(h, w3_ref[...],
                           preferred_element_type=jnp.float32) + b3_ref[...]


_tcC = pl.pallas_call(
    _tcC_body,
    grid=_GRID,
    in_specs=[
        _row_spec128,
        pl.BlockSpec((_RB, 16), lambda i: (i, 0)),
        _row_spec128,
        pl.BlockSpec((1, 16), lambda i: (0, 0)),
        pl.BlockSpec((16, 2), lambda i: (0, 0)),
        pl.BlockSpec((1, 2), lambda i: (0, 0)),
    ],
    out_specs=pl.BlockSpec((_RB, 2), lambda i: (i, 0)),
    out_shape=jax.ShapeDtypeStruct((N, 2), jnp.float32),
)


def kernel(x, edge_index, W1, b1, W2, b2, W3, b3):
    ei = edge_index.astype(jnp.int32)

    ones_deg = jnp.ones((CHUNK, DEGW), jnp.float32)
    zeros_deg = jnp.zeros((NPAD, DEGW), jnp.float32)
    zeros32 = jnp.zeros((NPAD, 32), jnp.float32)
    zeros16 = jnp.zeros((NPAD, 16), jnp.float32)

    deg2 = _deg_kernel(ei, ones_deg, zeros_deg)
    xw = _tcxw(x, W1)
    y1 = _tcscale(xw, deg2)
    s1 = _scat32(ei, y1, zeros32)
    y2 = _tcB(s1, xw, deg2, b1.reshape(1, 32), W2)
    s2 = _scat16(ei, y2, zeros16)
    out = _tcC(s2, y2, deg2, b2.reshape(1, 16), W3, b3.reshape(1, 2))
    return out


# speedup vs baseline: 2.2240x; 1.0729x over previous
import functools

import jax
import jax.numpy as jnp
from jax import lax
from jax.experimental import pallas as pl
from jax.experimental.pallas import tpu as pltpu
from jax.experimental.pallas import tpu_sc as plsc

N = 10000
E = 320000
NPAD = 10240
NACC = 10112
CHUNK = 1000
CPT = 10
EPT = CPT * CHUNK
TILES = 32
DEGW = 8
NBUF = 2
RPT = NPAD // 16
RPA = NACC // 16

_MESH = dict(core_axis_name="c", subcore_axis_name="s")
_SC_PARAMS = pltpu.CompilerParams(use_tc_tiling_on_sc=False)


@functools.partial(
    pl.kernel,
    out_type=jax.ShapeDtypeStruct((NACC, 128), jnp.float32),
    mesh=plsc.VectorSubcoreMesh(**_MESH),
    scratch_types=[
        pltpu.VMEM((EPT,), jnp.int32),
        pltpu.VMEM((CHUNK, DEGW), jnp.float32),
        pltpu.VMEM_SHARED((NACC, DEGW), jnp.float32),
        pltpu.SemaphoreType.DMA,
    ],
    compiler_params=_SC_PARAMS,
)
def _deg_kernel(ei_hbm, ones_hbm, zeros_hbm, out_hbm, dst_v, ones_v, acc_sh, sem):
    cid = lax.axis_index("c")
    sid = lax.axis_index("s")
    wid = cid * 16 + sid
    pltpu.sync_copy(ei_hbm.at[1, pl.ds(wid * EPT, EPT)], dst_v)
    pltpu.sync_copy(ones_hbm, ones_v)
    row0 = sid * RPA
    pltpu.sync_copy(zeros_hbm.at[pl.ds(row0, RPA)],
                    acc_sh.at[pl.ds(row0, RPA)])
    plsc.subcore_barrier()

    def fire(j, carry):
        pltpu.async_copy(ones_v, acc_sh.at[dst_v.at[pl.ds(j * CHUNK, CHUNK)]],
                         sem, add=True)
        return carry

    lax.fori_loop(0, CPT, fire, 0)

    def drain(j, carry):
        pltpu.make_async_copy(ones_v, acc_sh.at[dst_v.at[pl.ds(0, CHUNK)]],
                              sem).wait()
        return carry

    lax.fori_loop(0, CPT, drain, 0)
    plsc.subcore_barrier()
    pltpu.sync_copy(acc_sh.at[pl.ds(row0, RPA)],
                    out_hbm.at[pl.ds(row0, RPA), pl.ds(cid * DEGW, DEGW)])


def _make_scatter(D, staged):
    @functools.partial(
        pl.kernel,
        out_type=jax.ShapeDtypeStruct((NACC, 128), jnp.float32),
        mesh=plsc.VectorSubcoreMesh(**_MESH),
        scratch_types=(
            [pltpu.VMEM((EPT,), jnp.int32),
             pltpu.VMEM((EPT,), jnp.int32)]
            + [pltpu.VMEM((CHUNK, D), jnp.float32) for _ in range(NBUF)]
            + ([pltpu.VMEM((RPT, D), jnp.float32),
                pltpu.VMEM_SHARED((N, D), jnp.float32)] if staged else [])
            + [pltpu.VMEM_SHARED((NACC, D), jnp.float32)]
            + [pltpu.SemaphoreType.DMA for _ in range(2 * NBUF)]
        ),
        compiler_params=_SC_PARAMS,
    )
    def _scat(ei_hbm, y_hbm, zeros_hbm, out_hbm, src_v, dst_v, *rest):
        bufs = rest[:NBUF]
        nst = 2 if staged else 0
        if staged:
            stage_v, ysh = rest[NBUF:NBUF + 2]
        acc_sh = rest[NBUF + nst]
        gsem = rest[NBUF + nst + 1:NBUF + nst + 1 + NBUF]
        ssem = rest[NBUF + nst + 1 + NBUF:]
        cid = lax.axis_index("c")
        sid = lax.axis_index("s")
        wid = cid * 16 + sid
        pltpu.sync_copy(ei_hbm.at[0, pl.ds(wid * EPT, EPT)], src_v)
        pltpu.sync_copy(ei_hbm.at[1, pl.ds(wid * EPT, EPT)], dst_v)
        row0 = sid * RPA
        pltpu.sync_copy(zeros_hbm.at[pl.ds(row0, RPA)],
                        acc_sh.at[pl.ds(row0, RPA)])
        if staged:
            nstage = N - 15 * RPT
            srow0 = sid * RPT

            @pl.when(sid < 15)
            def _():
                pltpu.sync_copy(y_hbm.at[pl.ds(srow0, RPT), pl.ds(0, D)],
                                stage_v)
                pltpu.sync_copy(stage_v, ysh.at[pl.ds(srow0, RPT)])

            @pl.when(sid == 15)
            def _():
                pltpu.sync_copy(y_hbm.at[pl.ds(15 * RPT, nstage), pl.ds(0, D)],
                                stage_v.at[pl.ds(0, nstage)])
                pltpu.sync_copy(stage_v.at[pl.ds(0, nstage)],
                                ysh.at[pl.ds(15 * RPT, nstage)])

        plsc.subcore_barrier()

        def sidx(j):
            return src_v.at[pl.ds(j * CHUNK, CHUNK)]

        def didx(j):
            return dst_v.at[pl.ds(j * CHUNK, CHUNK)]

        def gsrc(j):
            return ysh.at[sidx(j)] if staged else y_hbm.at[sidx(j)]

        for b in range(NBUF):
            pltpu.async_copy(gsrc(b), bufs[b], gsem[b])

        def body(o, carry):
            j0 = o * NBUF
            for b in range(NBUF):
                j = j0 + b
                pltpu.make_async_copy(gsrc(j), bufs[b], gsem[b]).wait()
                pltpu.async_copy(bufs[b], acc_sh.at[didx(j)], ssem[b],
                                 add=True)
            for b in range(NBUF):
                j = j0 + b

                @pl.when(j + NBUF < CPT)
                def _():
                    pltpu.make_async_copy(bufs[b], acc_sh.at[didx(j)],
                                          ssem[b]).wait()
                    pltpu.async_copy(gsrc(j + NBUF), bufs[b], gsem[b])
            return carry

        lax.fori_loop(0, CPT // NBUF, body, 0)
        for b in range(NBUF):
            pltpu.make_async_copy(bufs[b], acc_sh.at[didx(CPT - NBUF + b)],
                                  ssem[b]).wait()
        plsc.subcore_barrier()
        pltpu.sync_copy(acc_sh.at[pl.ds(row0, RPA)],
                        out_hbm.at[pl.ds(row0, RPA), pl.ds(cid * D, D)])

    return _scat


_scat32 = _make_scatter(32, staged=False)
_scat16 = _make_scatter(16, staged=True)


_RB = 1000
_GRID = (N // _RB,)
_row_spec128 = pl.BlockSpec((_RB, 128), lambda i: (i, 0))


def _dis_block(deg_ref, off):
    deg = deg_ref[:, off:off + 1] + deg_ref[:, off + DEGW:off + DEGW + 1] + 1.0
    return lax.rsqrt(deg)


def _tcxw_body(x_ref, w_ref, xw_ref):
    xw_ref[...] = jnp.dot(x_ref[...], w_ref[...],
                          preferred_element_type=jnp.float32)


_tcxw = pl.pallas_call(
    _tcxw_body,
    grid=_GRID,
    in_specs=[
        _row_spec128,
        pl.BlockSpec((128, 32), lambda i: (0, 0)),
    ],
    out_specs=pl.BlockSpec((_RB, 32), lambda i: (i, 0)),
    out_shape=jax.ShapeDtypeStruct((N, 32), jnp.float32),
)


def _tcscale_body(xw_ref, deg_ref, y_ref):
    y_ref[...] = xw_ref[...] * _dis_block(deg_ref, 0)


_tcscale = pl.pallas_call(
    _tcscale_body,
    grid=_GRID,
    in_specs=[
        pl.BlockSpec((_RB, 32), lambda i: (i, 0)),
        _row_spec128,
    ],
    out_specs=pl.BlockSpec((_RB, 32), lambda i: (i, 0)),
    out_shape=jax.ShapeDtypeStruct((N, 32), jnp.float32),
)


def _tcB_body(s_ref, xw_ref, deg_ref, b1_ref, w2_ref, y2_ref):
    dis = _dis_block(deg_ref, 0)
    h = dis * (s_ref[:, 0:32] + s_ref[:, 32:64]) \
        + (dis * dis) * xw_ref[...] + b1_ref[...]
    h = jnp.maximum(h, 0.0)
    y2_ref[:, 0:16] = jnp.dot(h, w2_ref[...],
                              preferred_element_type=jnp.float32) * dis


_tcB = pl.pallas_call(
    _tcB_body,
    grid=_GRID,
    in_specs=[
        _row_spec128,
        pl.BlockSpec((_RB, 32), lambda i: (i, 0)),
        _row_spec128,
        pl.BlockSpec((1, 32), lambda i: (0, 0)),
        pl.BlockSpec((32, 16), lambda i: (0, 0)),
    ],
    out_specs=_row_spec128,
    out_shape=jax.ShapeDtypeStruct((N, 128), jnp.float32),
)


def _tcC_body(s_ref, y2_ref, deg_ref, b2_ref, w3_ref, b3_ref, out_ref):
    dis = _dis_block(deg_ref, 0)
    h = dis * (s_ref[:, 0:16] + s_ref[:, 16:32] + y2_ref[:, 0:16]) \
        + b2_ref[...]
    h = jnp.maximum(h, 0.0)
    out_ref[...] = jnp.dot(h, w3_ref[...],
                           preferred_element_type=jnp.float32) + b3_ref[...]


_tcC = pl.pallas_call(
    _tcC_body,
    grid=_GRID,
    in_specs=[
        _row_spec128,
        _row_spec128,
        _row_spec128,
        pl.BlockSpec((1, 16), lambda i: (0, 0)),
        pl.BlockSpec((16, 2), lambda i: (0, 0)),
        pl.BlockSpec((1, 2), lambda i: (0, 0)),
    ],
    out_specs=pl.BlockSpec((_RB, 2), lambda i: (i, 0)),
    out_shape=jax.ShapeDtypeStruct((N, 2), jnp.float32),
)


def kernel(x, edge_index, W1, b1, W2, b2, W3, b3):
    ei = edge_index.astype(jnp.int32)

    ones_deg = jnp.ones((CHUNK, DEGW), jnp.float32)
    zeros_deg = jnp.zeros((NACC, DEGW), jnp.float32)
    zeros32 = jnp.zeros((NACC, 32), jnp.float32)
    zeros16 = jnp.zeros((NACC, 16), jnp.float32)

    deg2 = _deg_kernel(ei, ones_deg, zeros_deg)
    xw = _tcxw(x, W1)
    y1 = _tcscale(xw, deg2)
    s1 = _scat32(ei, y1, zeros32)
    y2 = _tcB(s1, xw, deg2, b1.reshape(1, 32), W2)
    s2 = _scat16(ei, y2, zeros16)
    out = _tcC(s2, y2, deg2, b2.reshape(1, 16), W3, b3.reshape(1, 2))
    return out


# speedup vs baseline: 2.3228x; 1.0444x over previous
import functools

import jax
import jax.numpy as jnp
from jax import lax
from jax.experimental import pallas as pl
from jax.experimental.pallas import tpu as pltpu
from jax.experimental.pallas import tpu_sc as plsc

N = 10000
E = 320000
NPAD = 10240
NACC = 10112
CHUNK = 1000
CPT = 10
EPT = CPT * CHUNK
TILES = 32
DEGW = 8
NBUF = 2
RPT = NPAD // 16
RPA = NACC // 16

_MESH = dict(core_axis_name="c", subcore_axis_name="s")
_SC_PARAMS = pltpu.CompilerParams(use_tc_tiling_on_sc=False)


@functools.partial(
    pl.kernel,
    out_type=jax.ShapeDtypeStruct((NACC, 128), jnp.float32),
    mesh=plsc.VectorSubcoreMesh(**_MESH),
    scratch_types=[
        pltpu.VMEM((EPT,), jnp.int32),
        pltpu.VMEM((CHUNK, DEGW), jnp.float32),
        pltpu.VMEM_SHARED((NACC, DEGW), jnp.float32),
        pltpu.SemaphoreType.DMA,
    ],
    compiler_params=_SC_PARAMS,
)
def _deg_kernel(ei_hbm, ones_hbm, zeros_hbm, out_hbm, dst_v, ones_v, acc_sh, sem):
    cid = lax.axis_index("c")
    sid = lax.axis_index("s")
    wid = cid * 16 + sid
    pltpu.sync_copy(ei_hbm.at[1, pl.ds(wid * EPT, EPT)], dst_v)
    pltpu.sync_copy(ones_hbm, ones_v)
    row0 = sid * RPA
    pltpu.sync_copy(zeros_hbm.at[pl.ds(row0, RPA)],
                    acc_sh.at[pl.ds(row0, RPA)])
    plsc.subcore_barrier()

    def fire(j, carry):
        pltpu.async_copy(ones_v, acc_sh.at[dst_v.at[pl.ds(j * CHUNK, CHUNK)]],
                         sem, add=True)
        return carry

    lax.fori_loop(0, CPT, fire, 0)

    def drain(j, carry):
        pltpu.make_async_copy(ones_v, acc_sh.at[dst_v.at[pl.ds(0, CHUNK)]],
                              sem).wait()
        return carry

    lax.fori_loop(0, CPT, drain, 0)
    plsc.subcore_barrier()
    pltpu.sync_copy(acc_sh.at[pl.ds(row0, RPA)],
                    out_hbm.at[pl.ds(row0, RPA), pl.ds(cid * DEGW, DEGW)])


def _make_scatter(D, staged):
    @functools.partial(
        pl.kernel,
        out_type=jax.ShapeDtypeStruct((NACC, 128), jnp.float32),
        mesh=plsc.VectorSubcoreMesh(**_MESH),
        scratch_types=(
            [pltpu.VMEM((EPT,), jnp.int32),
             pltpu.VMEM((EPT,), jnp.int32)]
            + [pltpu.VMEM((CHUNK, D), jnp.float32) for _ in range(NBUF)]
            + ([pltpu.VMEM((RPT, D), jnp.float32),
                pltpu.VMEM_SHARED((N, D), jnp.float32)] if staged else [])
            + [pltpu.VMEM_SHARED((NACC, D), jnp.float32)]
            + [pltpu.SemaphoreType.DMA for _ in range(2 * NBUF)]
        ),
        compiler_params=_SC_PARAMS,
    )
    def _scat(ei_hbm, y_hbm, zeros_hbm, out_hbm, src_v, dst_v, *rest):
        bufs = rest[:NBUF]
        nst = 2 if staged else 0
        if staged:
            stage_v, ysh = rest[NBUF:NBUF + 2]
        acc_sh = rest[NBUF + nst]
        gsem = rest[NBUF + nst + 1:NBUF + nst + 1 + NBUF]
        ssem = rest[NBUF + nst + 1 + NBUF:]
        cid = lax.axis_index("c")
        sid = lax.axis_index("s")
        wid = cid * 16 + sid
        pltpu.sync_copy(ei_hbm.at[0, pl.ds(wid * EPT, EPT)], src_v)
        pltpu.sync_copy(ei_hbm.at[1, pl.ds(wid * EPT, EPT)], dst_v)
        row0 = sid * RPA
        pltpu.sync_copy(zeros_hbm.at[pl.ds(row0, RPA)],
                        acc_sh.at[pl.ds(row0, RPA)])
        if staged:
            nstage = N - 15 * RPT
            srow0 = sid * RPT

            @pl.when(sid < 15)
            def _():
                pltpu.sync_copy(y_hbm.at[pl.ds(srow0, RPT), pl.ds(0, D)],
                                stage_v)
                pltpu.sync_copy(stage_v, ysh.at[pl.ds(srow0, RPT)])

            @pl.when(sid == 15)
            def _():
                pltpu.sync_copy(y_hbm.at[pl.ds(15 * RPT, nstage), pl.ds(0, D)],
                                stage_v.at[pl.ds(0, nstage)])
                pltpu.sync_copy(stage_v.at[pl.ds(0, nstage)],
                                ysh.at[pl.ds(15 * RPT, nstage)])

        plsc.subcore_barrier()

        def sidx(j):
            return src_v.at[pl.ds(j * CHUNK, CHUNK)]

        def didx(j):
            return dst_v.at[pl.ds(j * CHUNK, CHUNK)]

        def gsrc(j):
            return ysh.at[sidx(j)] if staged else y_hbm.at[sidx(j)]

        for b in range(NBUF):
            pltpu.async_copy(gsrc(b), bufs[b], gsem[b])

        def body(o, carry):
            j0 = o * NBUF
            for b in range(NBUF):
                j = j0 + b
                pltpu.make_async_copy(gsrc(j), bufs[b], gsem[b]).wait()
                pltpu.async_copy(bufs[b], acc_sh.at[didx(j)], ssem[b],
                                 add=True)
            for b in range(NBUF):
                j = j0 + b

                @pl.when(j + NBUF < CPT)
                def _():
                    pltpu.make_async_copy(bufs[b], acc_sh.at[didx(j)],
                                          ssem[b]).wait()
                    pltpu.async_copy(gsrc(j + NBUF), bufs[b], gsem[b])
            return carry

        lax.fori_loop(0, CPT // NBUF, body, 0)
        for b in range(NBUF):
            pltpu.make_async_copy(bufs[b], acc_sh.at[didx(CPT - NBUF + b)],
                                  ssem[b]).wait()
        plsc.subcore_barrier()
        pltpu.sync_copy(acc_sh.at[pl.ds(row0, RPA)],
                        out_hbm.at[pl.ds(row0, RPA), pl.ds(cid * D, D)])

    return _scat


_scat32 = _make_scatter(32, staged=False)
_scat16 = _make_scatter(16, staged=True)


_RB = 2000
_GRID = (N // _RB,)
_row_spec128 = pl.BlockSpec((_RB, 128), lambda i: (i, 0))


def _dis_block(deg_ref, off):
    deg = deg_ref[:, off:off + 1] + deg_ref[:, off + DEGW:off + DEGW + 1] + 1.0
    return lax.rsqrt(deg)


def _tcxw_body(x_ref, w_ref, xw_ref):
    xw_ref[...] = jnp.dot(x_ref[...], w_ref[...],
                          preferred_element_type=jnp.float32)


_tcxw = pl.pallas_call(
    _tcxw_body,
    grid=_GRID,
    in_specs=[
        _row_spec128,
        pl.BlockSpec((128, 32), lambda i: (0, 0)),
    ],
    out_specs=pl.BlockSpec((_RB, 32), lambda i: (i, 0)),
    out_shape=jax.ShapeDtypeStruct((N, 32), jnp.float32),
)


def _tcscale_body(xw_ref, deg_ref, y_ref, dis8_ref):
    dis = _dis_block(deg_ref, 0)
    y_ref[...] = xw_ref[...] * dis
    dis8_ref[...] = jnp.broadcast_to(dis, (_RB, 8))


_tcscale = pl.pallas_call(
    _tcscale_body,
    grid=_GRID,
    in_specs=[
        pl.BlockSpec((_RB, 32), lambda i: (i, 0)),
        _row_spec128,
    ],
    out_specs=[pl.BlockSpec((_RB, 32), lambda i: (i, 0)),
               pl.BlockSpec((_RB, 8), lambda i: (i, 0))],
    out_shape=[jax.ShapeDtypeStruct((N, 32), jnp.float32),
               jax.ShapeDtypeStruct((N, 8), jnp.float32)],
)


def _tcB_body(s_ref, xw_ref, dis_ref, b1_ref, w2_ref, y2_ref):
    dis = dis_ref[:, 0:1]
    h = dis * (s_ref[:, 0:32] + s_ref[:, 32:64]) \
        + (dis * dis) * xw_ref[...] + b1_ref[...]
    h = jnp.maximum(h, 0.0)
    y2_ref[:, 0:16] = jnp.dot(h, w2_ref[...],
                              preferred_element_type=jnp.float32) * dis


_tcB = pl.pallas_call(
    _tcB_body,
    grid=_GRID,
    in_specs=[
        _row_spec128,
        pl.BlockSpec((_RB, 32), lambda i: (i, 0)),
        pl.BlockSpec((_RB, 8), lambda i: (i, 0)),
        pl.BlockSpec((1, 32), lambda i: (0, 0)),
        pl.BlockSpec((32, 16), lambda i: (0, 0)),
    ],
    out_specs=_row_spec128,
    out_shape=jax.ShapeDtypeStruct((N, 128), jnp.float32),
)


def _tcC_body(s_ref, y2_ref, dis_ref, b2_ref, w3_ref, b3_ref, out_ref):
    dis = dis_ref[:, 0:1]
    h = dis * (s_ref[:, 0:16] + s_ref[:, 16:32] + y2_ref[:, 0:16]) \
        + b2_ref[...]
    h = jnp.maximum(h, 0.0)
    out_ref[...] = jnp.dot(h, w3_ref[...],
                           preferred_element_type=jnp.float32) + b3_ref[...]


_tcC = pl.pallas_call(
    _tcC_body,
    grid=_GRID,
    in_specs=[
        _row_spec128,
        _row_spec128,
        pl.BlockSpec((_RB, 8), lambda i: (i, 0)),
        pl.BlockSpec((1, 16), lambda i: (0, 0)),
        pl.BlockSpec((16, 2), lambda i: (0, 0)),
        pl.BlockSpec((1, 2), lambda i: (0, 0)),
    ],
    out_specs=pl.BlockSpec((_RB, 2), lambda i: (i, 0)),
    out_shape=jax.ShapeDtypeStruct((N, 2), jnp.float32),
)


def kernel(x, edge_index, W1, b1, W2, b2, W3, b3):
    ei = edge_index.astype(jnp.int32)

    ones_deg = jnp.ones((CHUNK, DEGW), jnp.float32)
    zeros_deg = jnp.zeros((NACC, DEGW), jnp.float32)
    zeros32 = jnp.zeros((NACC, 32), jnp.float32)
    zeros16 = jnp.zeros((NACC, 16), jnp.float32)

    deg2 = _deg_kernel(ei, ones_deg, zeros_deg)
    xw = _tcxw(x, W1)
    y1, dis8 = _tcscale(xw, deg2)
    s1 = _scat32(ei, y1, zeros32)
    y2 = _tcB(s1, xw, dis8, b1.reshape(1, 32), W2)
    s2 = _scat16(ei, y2, zeros16)
    out = _tcC(s2, y2, dis8, b2.reshape(1, 16), W3, b3.reshape(1, 2))
    return out
